# baseline (speedup 1.0000x reference)
import functools

import jax
import jax.numpy as jnp
import numpy as np
from jax.experimental import pallas as pl
from jax.experimental.pallas import tpu as pltpu

_HEAD_ORDER = ['role_select', 'settler', 'builder', 'mayor', 'craftsman', 'trader', 'captain']
_HEAD_HIDDEN = [512, 256, 512, 512, 128, 256, 512]
_PHASE_TO_HEADIDX = np.array([1, 3, 2, 4, 5, 6, 6, 0, 0], dtype=np.int32)
_OFFS = np.concatenate([[0], np.cumsum(_HEAD_HIDDEN)])
_HSUM = int(_OFFS[-1])


def _bdot(a, b):
    return jax.lax.dot(a.astype(jnp.bfloat16), b.astype(jnp.bfloat16),
                       preferred_element_type=jnp.float32)


def _normalize(x, eps=1e-5):
    m = jnp.mean(x, axis=-1, keepdims=True)
    v = jnp.mean((x - m) ** 2, axis=-1, keepdims=True)
    return (x - m) * jax.lax.rsqrt(v + eps)


def _fused_body(
    x_ref, ph_ref, act_ref, amask_ref,
    pe_tab_ref, wx_ref, wp_ref, be_ref, ge_ref, bee_ref,
    bw1_ref, bb1_ref, bw2_ref, bb2_ref,
    cw1_ref, cb1_ref, cw2_ref, cb2_ref,
    hw1_ref, hb1_ref, hw2_ref, hb2_ref, p2h_ref,
    logp_ref, ent_ref, val_ref,
):
    f32 = jnp.float32
    blk = x_ref.shape[0]

    ph = ph_ref[...]
    iota9 = jax.lax.broadcasted_iota(jnp.int32, (blk, 9), 1)
    oh9 = (ph == iota9).astype(f32)
    pe = jnp.dot(oh9, pe_tab_ref[...])

    u = _bdot(x_ref[...], wx_ref[...]) + _bdot(pe, wp_ref[...]) + be_ref[...]
    h = jax.nn.relu(_normalize(u) * ge_ref[...] + bee_ref[...])

    for i in range(3):
        t = _normalize(h)
        t = jax.nn.relu(_bdot(t, bw1_ref[i]) + bb1_ref[i])
        t = jax.nn.relu(_bdot(t, bw2_ref[i]) + bb2_ref[i])
        h = h + t

    nrm = _normalize(h)

    v = jax.nn.relu(_bdot(nrm, cw1_ref[...]) + cb1_ref[...])
    val_ref[...] = _bdot(v, cw2_ref[...]) + cb2_ref[...]

    h1 = jax.nn.relu(_bdot(nrm, hw1_ref[...]) + hb1_ref[...])

    hid = jnp.dot(oh9, p2h_ref[...]).astype(jnp.int32)
    cols = jax.lax.broadcasted_iota(jnp.int32, (1, _HSUM), 1)
    seg = jnp.zeros((1, _HSUM), jnp.int32)
    for off in _OFFS[1:-1]:
        seg = seg + (cols >= int(off)).astype(jnp.int32)
    h1m = h1 * (seg == hid).astype(f32)

    logits = _bdot(h1m, hw2_ref[...])
    iota7 = jax.lax.broadcasted_iota(jnp.int32, (blk, 7), 1)
    oh7 = (iota7 == hid).astype(f32)
    logits = logits + jnp.dot(oh7, hb2_ref[...])

    masked = jnp.where(amask_ref[...] > 0.5, logits, f32(-1e8))
    mx = jnp.max(masked, axis=-1, keepdims=True)
    z = masked - mx
    ez = jnp.exp(z)
    s = jnp.sum(ez, axis=-1, keepdims=True)
    logp = z - jnp.log(s)

    act = act_ref[...]
    iota_a = jax.lax.broadcasted_iota(jnp.int32, (blk, logits.shape[1]), 1)
    oh_a = (act == iota_a).astype(f32)
    logp_ref[...] = jnp.sum(logp * oh_a, axis=-1, keepdims=True)
    probs = ez / s
    ent_ref[...] = -jnp.sum(probs * logp, axis=-1, keepdims=True)


@functools.partial(jax.jit, static_argnames=())
def _run(x, action_mask, phase_ids, action, params):
    B, OBS = x.shape
    ACT = action_mask.shape[1]
    H = params['embed']['W'].shape[1]
    PE = params['phase_embed'].shape[1]
    BLK = 512
    nb = B // BLK

    e = params['embed']
    wx = e['W'][:OBS]
    wp = e['W'][OBS:]

    bw1 = jnp.stack([blk['g'][:, None] * blk['W1'] for blk in params['blocks']])
    bb1 = jnp.stack([blk['b1'] + blk['be'] @ blk['W1'] for blk in params['blocks']])
    bw2 = jnp.stack([blk['W2'] for blk in params['blocks']])
    bb2 = jnp.stack([blk['b2'] for blk in params['blocks']])

    c = params['critic']
    cw1 = c['g'][:, None] * c['W1']
    cb1 = c['b1'] + c['be'] @ c['W1']

    hp = params['heads']
    hw1 = jnp.concatenate(
        [hp[n]['g'][:, None] * hp[n]['W1'] for n in _HEAD_ORDER], axis=1)
    hb1 = jnp.concatenate(
        [hp[n]['b1'] + hp[n]['be'] @ hp[n]['W1'] for n in _HEAD_ORDER])
    hw2 = jnp.concatenate([hp[n]['W2'] for n in _HEAD_ORDER], axis=0)
    hb2 = jnp.stack([hp[n]['b2'] for n in _HEAD_ORDER])

    ph2 = phase_ids.astype(jnp.int32).reshape(B, 1)
    act2 = action.astype(jnp.int32).reshape(B, 1)

    row_spec = lambda w: pl.BlockSpec((BLK, w), lambda i: (i, 0))
    full = lambda *shape: pl.BlockSpec(shape, lambda i: (0,) * len(shape))

    out_shapes = [
        jax.ShapeDtypeStruct((B, 1), jnp.float32),
        jax.ShapeDtypeStruct((B, 1), jnp.float32),
        jax.ShapeDtypeStruct((B, 1), jnp.float32),
    ]
    logp, ent, val = pl.pallas_call(
        _fused_body,
        grid=(nb,),
        in_specs=[
            row_spec(OBS), row_spec(1), row_spec(1), row_spec(ACT),
            full(9, PE), full(OBS, H), full(PE, H), full(H), full(H), full(H),
            full(3, H, H), full(3, H), full(3, H, H), full(3, H),
            full(H, H), full(H), full(H, 1), full(1),
            full(H, _HSUM), full(_HSUM), full(_HSUM, ACT), full(7, ACT),
            full(9, 1),
        ],
        out_specs=[row_spec(1), row_spec(1), row_spec(1)],
        out_shape=out_shapes,
    )(
        x, ph2, act2, action_mask,
        params['phase_embed'], wx, wp, e['b'], e['g'], e['be'],
        bw1, bb1, bw2, bb2,
        cw1, cb1, c['W2'], c['b2'],
        hw1, hb1, hw2, hb2,
        jnp.asarray(_PHASE_TO_HEADIDX.astype(np.float32)[:, None]),
    )
    return action, logp[:, 0], ent[:, 0], val


def kernel(x, action_mask, phase_ids, action, params):
    return _run(x, action_mask, phase_ids, action, params)


# speedup vs baseline: 1.0615x; 1.0615x over previous
import functools

import jax
import jax.numpy as jnp
import numpy as np
from jax.experimental import pallas as pl
from jax.experimental.pallas import tpu as pltpu

_HEAD_ORDER = ['role_select', 'settler', 'builder', 'mayor', 'craftsman', 'trader', 'captain']
_HEAD_HIDDEN = [512, 256, 512, 512, 128, 256, 512]
_PHASE_TO_HEADIDX = np.array([1, 3, 2, 4, 5, 6, 6, 0, 0], dtype=np.int32)
_OFFS = np.concatenate([[0], np.cumsum(_HEAD_HIDDEN)])
_HSUM = int(_OFFS[-1])


def _bdot(a, b):
    return jax.lax.dot(a.astype(jnp.bfloat16), b,
                       preferred_element_type=jnp.float32)


def _bdot16(a, b):
    return jax.lax.dot(a, b, preferred_element_type=jnp.float32)


def _normalize(x, eps=1e-5):
    m = jnp.mean(x, axis=-1, keepdims=True)
    v = jnp.mean(x * x, axis=-1, keepdims=True) - m * m
    return (x - m) * jax.lax.rsqrt(v + eps)


def _fused_body(
    x_ref, ph_ref, act_ref, amask_ref,
    pe_tab_ref, wx_ref, wp_ref, be_ref, ge_ref, bee_ref,
    bw1_ref, bb1_ref, bw2_ref, bb2_ref,
    cw1_ref, cb1_ref, cw2_ref, cb2_ref,
    hw1_ref, hb1_ref, hw2_ref, hb2_ref, p2h_ref,
    logp_ref, ent_ref, val_ref,
):
    f32 = jnp.float32
    blk = x_ref.shape[0]

    ph = ph_ref[...]
    iota9 = jax.lax.broadcasted_iota(jnp.int32, (blk, 9), 1)
    oh9 = (ph == iota9).astype(f32)
    pe = jnp.dot(oh9, pe_tab_ref[...])

    u = _bdot(x_ref[...], wx_ref[...]) + _bdot(pe, wp_ref[...]) + be_ref[...]
    h = jax.nn.relu(_normalize(u) * ge_ref[...] + bee_ref[...])

    for i in range(3):
        t = _normalize(h)
        t = jax.nn.relu(_bdot(t, bw1_ref[i]) + bb1_ref[i])
        t = jax.nn.relu(_bdot(t, bw2_ref[i]) + bb2_ref[i])
        h = h + t

    nrm = _normalize(h).astype(jnp.bfloat16)

    v = jax.nn.relu(_bdot16(nrm, cw1_ref[...]) + cb1_ref[...])
    val_ref[...] = _bdot(v, cw2_ref[...]) + cb2_ref[...]

    h1 = _bdot16(nrm, hw1_ref[...]) + hb1_ref[...]

    hid = jnp.dot(oh9, p2h_ref[...]).astype(jnp.int32)
    cols = jax.lax.broadcasted_iota(jnp.int32, (1, _HSUM), 1)
    seg = jnp.zeros((1, _HSUM), jnp.int32)
    for off in _OFFS[1:-1]:
        seg = seg + (cols >= int(off)).astype(jnp.int32)
    h1m = jnp.where(seg == hid, jax.nn.relu(h1).astype(jnp.bfloat16),
                    jnp.bfloat16(0))

    logits = _bdot16(h1m, hw2_ref[...])
    iota7 = jax.lax.broadcasted_iota(jnp.int32, (blk, 7), 1)
    oh7 = (iota7 == hid).astype(f32)
    logits = logits + jnp.dot(oh7, hb2_ref[...])

    masked = jnp.where(amask_ref[...] > 0.5, logits, f32(-1e8))
    mx = jnp.max(masked, axis=-1, keepdims=True)
    z = masked - mx
    ez = jnp.exp(z)
    s = jnp.sum(ez, axis=-1, keepdims=True)
    logp = z - jnp.log(s)

    act = act_ref[...]
    iota_a = jax.lax.broadcasted_iota(jnp.int32, (blk, logits.shape[1]), 1)
    oh_a = (act == iota_a).astype(f32)
    logp_ref[...] = jnp.sum(logp * oh_a, axis=-1, keepdims=True)
    probs = ez / s
    ent_ref[...] = -jnp.sum(probs * logp, axis=-1, keepdims=True)


@functools.partial(jax.jit, static_argnames=())
def _run(x, action_mask, phase_ids, action, params):
    B, OBS = x.shape
    ACT = action_mask.shape[1]
    H = params['embed']['W'].shape[1]
    PE = params['phase_embed'].shape[1]
    BLK = 512
    nb = B // BLK

    bf16 = jnp.bfloat16
    e = params['embed']
    wx = e['W'][:OBS].astype(bf16)
    wp = e['W'][OBS:].astype(bf16)

    bw1 = jnp.stack([blk['g'][:, None] * blk['W1'] for blk in params['blocks']]).astype(bf16)
    bb1 = jnp.stack([blk['b1'] + blk['be'] @ blk['W1'] for blk in params['blocks']])
    bw2 = jnp.stack([blk['W2'] for blk in params['blocks']]).astype(bf16)
    bb2 = jnp.stack([blk['b2'] for blk in params['blocks']])

    c = params['critic']
    cw1 = (c['g'][:, None] * c['W1']).astype(bf16)
    cb1 = c['b1'] + c['be'] @ c['W1']

    hp = params['heads']
    hw1 = jnp.concatenate(
        [hp[n]['g'][:, None] * hp[n]['W1'] for n in _HEAD_ORDER], axis=1).astype(bf16)
    hb1 = jnp.concatenate(
        [hp[n]['b1'] + hp[n]['be'] @ hp[n]['W1'] for n in _HEAD_ORDER])
    hw2 = jnp.concatenate([hp[n]['W2'] for n in _HEAD_ORDER], axis=0).astype(bf16)
    hb2 = jnp.stack([hp[n]['b2'] for n in _HEAD_ORDER])

    ph2 = phase_ids.astype(jnp.int32).reshape(B, 1)
    act2 = action.astype(jnp.int32).reshape(B, 1)

    row_spec = lambda w: pl.BlockSpec((BLK, w), lambda i: (i, 0))
    full = lambda *shape: pl.BlockSpec(shape, lambda i: (0,) * len(shape))

    out_shapes = [
        jax.ShapeDtypeStruct((B, 1), jnp.float32),
        jax.ShapeDtypeStruct((B, 1), jnp.float32),
        jax.ShapeDtypeStruct((B, 1), jnp.float32),
    ]
    logp, ent, val = pl.pallas_call(
        _fused_body,
        grid=(nb,),
        in_specs=[
            row_spec(OBS), row_spec(1), row_spec(1), row_spec(ACT),
            full(9, PE), full(OBS, H), full(PE, H), full(H), full(H), full(H),
            full(3, H, H), full(3, H), full(3, H, H), full(3, H),
            full(H, H), full(H), full(H, 1), full(1),
            full(H, _HSUM), full(_HSUM), full(_HSUM, ACT), full(7, ACT),
            full(9, 1),
        ],
        out_specs=[row_spec(1), row_spec(1), row_spec(1)],
        out_shape=out_shapes,
    )(
        x, ph2, act2, action_mask,
        params['phase_embed'], wx, wp, e['b'], e['g'], e['be'],
        bw1, bb1, bw2, bb2,
        cw1, cb1, c['W2'].astype(bf16), c['b2'],
        hw1, hb1, hw2, hb2,
        jnp.asarray(_PHASE_TO_HEADIDX.astype(np.float32)[:, None]),
    )
    return action, logp[:, 0], ent[:, 0], val


def kernel(x, action_mask, phase_ids, action, params):
    return _run(x, action_mask, phase_ids, action, params)
